# Initial kernel scaffold; baseline (speedup 1.0000x reference)
import functools

import jax
import jax.numpy as jnp
from jax import lax
from jax.experimental import pallas as pl
from jax.experimental.pallas import tpu as pltpu
from jax.experimental.pallas import tpu_sc as plsc

N = 10000
D = 128
E = 320000
NPAD = 10240
NC = 2
NS = 16
L = 16
NW = NC * NS
CHUNK = 128
EPT = NPAD
NCHUNK = EPT // CHUNK
EPAD = NW * EPT
RPT = NPAD // NS
SIGMA = 1.0

_mesh = plsc.VectorSubcoreMesh(
    core_axis_name="c", subcore_axis_name="s", num_cores=NC, num_subcores=NS
)


def _sc_agg_body(with_cnt, h_hbm, srcg, dstg, zrows, zcnt, out, cntp,
                 src_v, dst_v, rows_v, cnt_v, acc_sh, sem0, sem1):
    c = lax.axis_index("c")
    s = lax.axis_index("s")
    wid = c * NS + s

    pltpu.sync_copy(srcg.at[wid], src_v)
    pltpu.sync_copy(dstg.at[wid], dst_v)

    pltpu.sync_copy(zrows, rows_v.at[0])
    for k in range(RPT // CHUNK):
        pltpu.sync_copy(rows_v.at[0],
                        acc_sh.at[pl.ds(s * RPT + k * CHUNK, CHUNK)])

    if with_cnt:
        pltpu.sync_copy(zcnt, cnt_v)
        ones = jnp.ones((L,), jnp.float32)

        @pl.loop(0, NCHUNK)
        def _(j):
            @pl.loop(0, CHUNK // L)
            def _(q):
                idx = dst_v[j, pl.ds(q * L, L)]
                plsc.addupdate_scatter(cnt_v, [idx], ones)

        pltpu.sync_copy(cnt_v, cntp.at[wid])

    plsc.subcore_barrier()

    pltpu.async_copy(h_hbm.at[src_v.at[0]], rows_v.at[0], sem0)

    @pl.loop(0, NCHUNK, step=2)
    def _(jj):
        pltpu.make_async_copy(h_hbm.at[pl.ds(0, CHUNK)], rows_v.at[0],
                              sem0).wait()
        pltpu.async_copy(h_hbm.at[src_v.at[jj + 1]], rows_v.at[1], sem1)
        pltpu.sync_copy(rows_v.at[0], acc_sh.at[dst_v.at[jj]], add=True)
        pltpu.make_async_copy(h_hbm.at[pl.ds(0, CHUNK)], rows_v.at[1],
                              sem1).wait()

        @pl.when(jj + 2 < NCHUNK)
        def _():
            pltpu.async_copy(h_hbm.at[src_v.at[jj + 2]], rows_v.at[0], sem0)

        pltpu.sync_copy(rows_v.at[1], acc_sh.at[dst_v.at[jj + 1]], add=True)

    plsc.subcore_barrier()

    for k in range(RPT // CHUNK):
        off = s * RPT + k * CHUNK
        pltpu.sync_copy(acc_sh.at[pl.ds(off, CHUNK)], rows_v.at[0])
        pltpu.sync_copy(rows_v.at[0], out.at[c, pl.ds(off, CHUNK)])


_sc_scratch = [
    pltpu.VMEM((NCHUNK, CHUNK), jnp.int32),
    pltpu.VMEM((NCHUNK, CHUNK), jnp.int32),
    pltpu.VMEM((2, CHUNK, D), jnp.float32),
    pltpu.VMEM((NPAD,), jnp.float32),
    pltpu.VMEM_SHARED((NPAD, D), jnp.float32),
    pltpu.SemaphoreType.DMA,
    pltpu.SemaphoreType.DMA,
]

_sc_agg_cnt = pl.kernel(
    functools.partial(_sc_agg_body, True),
    out_type=[
        jax.ShapeDtypeStruct((NC, NPAD, D), jnp.float32),
        jax.ShapeDtypeStruct((NW, NPAD), jnp.float32),
    ],
    mesh=_mesh,
    scratch_types=_sc_scratch,
)


def _sc_agg_nocnt_body(h_hbm, srcg, dstg, zrows, out,
                       src_v, dst_v, rows_v, cnt_v, acc_sh, sem0, sem1):
    _sc_agg_body(False, h_hbm, srcg, dstg, zrows, None, out, None,
                 src_v, dst_v, rows_v, cnt_v, acc_sh, sem0, sem1)


_sc_agg = pl.kernel(
    _sc_agg_nocnt_body,
    out_type=jax.ShapeDtypeStruct((NC, NPAD, D), jnp.float32),
    mesh=_mesh,
    scratch_types=_sc_scratch,
)


def _emb_body(x_ref, w_ref, b_ref, o_ref):
    o_ref[...] = (
        jnp.dot(x_ref[...], w_ref[...], preferred_element_type=jnp.float32)
        + b_ref[...]
    )


def _layer_core(h, tot, rinv, w_ref, b_ref, g_ref, be_ref):
    c = tot[0:N] * rinv
    out = (
        jnp.dot(h, w_ref[0:D], preferred_element_type=jnp.float32)
        + jnp.dot(c, w_ref[D:2 * D], preferred_element_type=jnp.float32)
        + b_ref[...]
    )
    nrm = jnp.sqrt(jnp.sum(out * out, axis=1, keepdims=True))
    out = out / jnp.maximum(nrm, 1e-12)
    out = jnp.maximum(out, 0.0)
    mu = jnp.mean(out, axis=0, keepdims=True)
    var = jnp.mean((out - mu) ** 2, axis=0, keepdims=True)
    out = g_ref[...] * (out - mu) / jnp.sqrt(var + 1e-5) + be_ref[...]
    return h + out


def _l0_body(h_ref, parts_ref, cntt_ref, w_ref, b_ref, g_ref, be_ref,
             o_ref, rinv_ref):
    cnt = jnp.sum(cntt_ref[...], axis=1, keepdims=True)
    rinv = 1.0 / jnp.maximum(cnt, 1.0)
    rinv_ref[...] = rinv
    tot = parts_ref[0] + parts_ref[1]
    o_ref[...] = _layer_core(h_ref[...], tot, rinv[0:N],
                             w_ref, b_ref, g_ref, be_ref)


def _lmid_body(h_ref, parts_ref, rinv_ref, w_ref, b_ref, g_ref, be_ref,
               o_ref):
    tot = parts_ref[0] + parts_ref[1]
    o_ref[...] = _layer_core(h_ref[...], tot, rinv_ref[0:N],
                             w_ref, b_ref, g_ref, be_ref)


def _l1_body(h_ref, parts_ref, rinv_ref, w_ref, b_ref, g_ref, be_ref,
             wa_ref, ba_ref, o_ref, s_ref):
    tot = parts_ref[0] + parts_ref[1]
    hn = _layer_core(h_ref[...], tot, rinv_ref[0:N],
                     w_ref, b_ref, g_ref, be_ref)
    o_ref[...] = hn
    z = (jnp.dot(hn, wa_ref[...], preferred_element_type=jnp.float32)
         + ba_ref[...]) / SIGMA
    m = jnp.max(z, axis=-1, keepdims=True)
    ez = jnp.exp(z - m)
    s_ref[...] = ez / jnp.sum(ez, axis=-1, keepdims=True)


def _l3_body(h_ref, parts_ref, rinv_ref, w_ref, b_ref, g_ref, be_ref,
             w0_ref, b0_ref, w1_ref, b1_ref, w2_ref, b2_ref, logits_ref):
    tot = parts_ref[0] + parts_ref[1]
    hn = _layer_core(h_ref[...], tot, rinv_ref[0:N],
                     w_ref, b_ref, g_ref, be_ref)
    hg = jnp.mean(hn, axis=0, keepdims=True)
    z = jnp.maximum(
        jnp.dot(hg, w0_ref[...], preferred_element_type=jnp.float32)
        + b0_ref[...], 0.0)
    z = jnp.maximum(
        jnp.dot(z, w1_ref[...], preferred_element_type=jnp.float32)
        + b1_ref[...], 0.0)
    logits_ref[...] = (
        jnp.dot(z, w2_ref[...], preferred_element_type=jnp.float32)
        + b2_ref[...])


_f32 = jnp.float32
_emb_tc = pl.pallas_call(
    _emb_body, out_shape=jax.ShapeDtypeStruct((N, D), _f32))
_l0_tc = pl.pallas_call(
    _l0_body,
    out_shape=[jax.ShapeDtypeStruct((N, D), _f32),
               jax.ShapeDtypeStruct((NPAD, 1), _f32)])
_lmid_tc = pl.pallas_call(
    _lmid_body, out_shape=jax.ShapeDtypeStruct((N, D), _f32))
_l1_tc = pl.pallas_call(
    _l1_body,
    out_shape=[jax.ShapeDtypeStruct((N, D), _f32),
               jax.ShapeDtypeStruct((N, 32), _f32)])
_l3_tc = pl.pallas_call(
    _l3_body, out_shape=jax.ShapeDtypeStruct((1, 10), _f32))


def kernel(x, e, edge_index, params):
    del e
    src = edge_index[0].astype(jnp.int32)
    dst = edge_index[1].astype(jnp.int32)
    pad = EPAD - E
    srcg = jnp.concatenate([src, jnp.zeros((pad,), jnp.int32)])
    dstg = jnp.concatenate([dst, jnp.full((pad,), N, jnp.int32)])
    srcg = srcg.reshape(NW, NCHUNK, CHUNK)
    dstg = dstg.reshape(NW, NCHUNK, CHUNK)
    zrows = jnp.zeros((CHUNK, D), _f32)
    zcnt = jnp.zeros((NPAD,), _f32)

    p = params

    def r2(v):
        return v.reshape(1, -1)

    h = _emb_tc(x, p['emb']['W'], r2(p['emb']['b']))

    parts, cntp = _sc_agg_cnt(h, srcg, dstg, zrows, zcnt)
    cntt = cntp.T
    h, rinv = _l0_tc(h, parts, cntt, p['l0']['W'], r2(p['l0']['b']),
                     r2(p['l0']['gamma']), r2(p['l0']['beta']))

    parts = _sc_agg(h, srcg, dstg, zrows)
    h, s = _l1_tc(h, parts, rinv, p['l1']['W'], r2(p['l1']['b']),
                  r2(p['l1']['gamma']), r2(p['l1']['beta']),
                  p['assign']['W'], r2(p['assign']['b']))

    parts = _sc_agg(h, srcg, dstg, zrows)
    h = _lmid_tc(h, parts, rinv, p['l2']['W'], r2(p['l2']['b']),
                 r2(p['l2']['gamma']), r2(p['l2']['beta']))

    parts = _sc_agg(h, srcg, dstg, zrows)
    logits = _l3_tc(h, parts, rinv, p['l3']['W'], r2(p['l3']['b']),
                    r2(p['l3']['gamma']), r2(p['l3']['beta']),
                    p['mlp0']['W'], r2(p['mlp0']['b']),
                    p['mlp1']['W'], r2(p['mlp1']['b']),
                    p['mlp2']['W'], r2(p['mlp2']['b']))
    return (logits, s)


# speedup vs baseline: 3.4458x; 3.4458x over previous
import dataclasses
import functools

import jax
import jax.numpy as jnp
from jax import lax
from jax.experimental import pallas as pl
from jax.experimental.pallas import tpu as pltpu
from jax.experimental.pallas import tpu_sc as plsc

N = 10000
D = 128
E = 320000
NPAD = 10240
NC = 2
NS = 16
L = 16
NW = NC * NS
CHUNK = 128
EPT = NPAD
NCHUNK = EPT // CHUNK
EPAD = NW * EPT
RPT = NPAD // NS
NSTAGE = 5
CPS = NCHUNK // NSTAGE
SIGMA = 1.0

def _sc_agg_body(with_cnt, h_hbm, srcg, dstg, zrows, zcnt, out, cntp,
                 src_v, dst_v, rows_v, cnt_v, acc_sh, sem0, sem1):
    c = lax.axis_index("c")
    s = lax.axis_index("s")
    wid = c * NS + s

    pltpu.sync_copy(zrows, rows_v.at[0])
    for k in range(RPT // CHUNK):
        pltpu.sync_copy(rows_v.at[0],
                        acc_sh.at[pl.ds(s * RPT + k * CHUNK, CHUNK)])

    if with_cnt:
        pltpu.sync_copy(zcnt, cnt_v)
        ones = jnp.ones((L,), jnp.float32)
        for st in range(NSTAGE):
            pltpu.sync_copy(dstg.at[wid, pl.ds(st * CPS, CPS)], dst_v)

            @pl.loop(0, CPS)
            def _(j):
                @pl.loop(0, CHUNK // L)
                def _(q):
                    idx = dst_v[j, pl.ds(q * L, L)]
                    plsc.addupdate_scatter(cnt_v, [idx], ones)

        pltpu.sync_copy(cnt_v, cntp.at[wid])

    plsc.subcore_barrier()

    for st in range(NSTAGE):
        pltpu.sync_copy(srcg.at[wid, pl.ds(st * CPS, CPS)], src_v)
        pltpu.sync_copy(dstg.at[wid, pl.ds(st * CPS, CPS)], dst_v)
        pltpu.async_copy(h_hbm.at[src_v.at[0]], rows_v.at[0], sem0)

        @pl.loop(0, CPS, step=2)
        def _(jj):
            pltpu.make_async_copy(h_hbm.at[pl.ds(0, CHUNK)], rows_v.at[0],
                                  sem0).wait()
            pltpu.async_copy(h_hbm.at[src_v.at[jj + 1]], rows_v.at[1], sem1)
            pltpu.sync_copy(rows_v.at[0], acc_sh.at[dst_v.at[jj]], add=True)
            pltpu.make_async_copy(h_hbm.at[pl.ds(0, CHUNK)], rows_v.at[1],
                                  sem1).wait()

            @pl.when(jj + 2 < CPS)
            def _():
                pltpu.async_copy(h_hbm.at[src_v.at[jj + 2]], rows_v.at[0],
                                 sem0)

            pltpu.sync_copy(rows_v.at[1], acc_sh.at[dst_v.at[jj + 1]],
                            add=True)

    plsc.subcore_barrier()

    for k in range(RPT // CHUNK):
        off = s * RPT + k * CHUNK
        pltpu.sync_copy(acc_sh.at[pl.ds(off, CHUNK)], rows_v.at[0])
        pltpu.sync_copy(rows_v.at[0], out.at[c, pl.ds(off, CHUNK)])


def _sc_agg_nocnt_body(h_hbm, srcg, dstg, zrows, out,
                       src_v, dst_v, rows_v, cnt_v, acc_sh, sem0, sem1):
    _sc_agg_body(False, h_hbm, srcg, dstg, zrows, None, out, None,
                 src_v, dst_v, rows_v, cnt_v, acc_sh, sem0, sem1)


@functools.cache
def _sc_kernels():
    mesh = plsc.VectorSubcoreMesh(
        core_axis_name="c", subcore_axis_name="s",
        num_cores=NC, num_subcores=NS)
    scratch = [
        pltpu.VMEM((CPS, CHUNK), jnp.int32),
        pltpu.VMEM((CPS, CHUNK), jnp.int32),
        pltpu.VMEM((2, CHUNK, D), jnp.float32),
        pltpu.VMEM((NPAD,), jnp.float32),
        pltpu.VMEM_SHARED((NPAD, D), jnp.float32),
        pltpu.SemaphoreType.DMA,
        pltpu.SemaphoreType.DMA,
    ]
    cp = pltpu.CompilerParams()
    if "needs_layout_passes" in pltpu.CompilerParams.__dataclass_fields__:
        cp = dataclasses.replace(cp, needs_layout_passes=False)
    agg_cnt = pl.kernel(
        functools.partial(_sc_agg_body, True),
        out_type=[
            jax.ShapeDtypeStruct((NC, NPAD, D), jnp.float32),
            jax.ShapeDtypeStruct((NW, NPAD), jnp.float32),
        ],
        mesh=mesh,
        scratch_types=scratch,
        compiler_params=cp,
    )
    agg = pl.kernel(
        _sc_agg_nocnt_body,
        out_type=jax.ShapeDtypeStruct((NC, NPAD, D), jnp.float32),
        mesh=mesh,
        scratch_types=scratch,
        compiler_params=cp,
    )
    return agg_cnt, agg


def _emb_body(x_ref, w_ref, b_ref, o_ref):
    o_ref[...] = (
        jnp.dot(x_ref[...], w_ref[...], preferred_element_type=jnp.float32)
        + b_ref[...]
    )


def _layer_core(h, tot, rinv, w_ref, b_ref, g_ref, be_ref):
    c = tot[0:N] * rinv
    out = (
        jnp.dot(h, w_ref[0:D], preferred_element_type=jnp.float32)
        + jnp.dot(c, w_ref[D:2 * D], preferred_element_type=jnp.float32)
        + b_ref[...]
    )
    nrm = jnp.sqrt(jnp.sum(out * out, axis=1, keepdims=True))
    out = out / jnp.maximum(nrm, 1e-12)
    out = jnp.maximum(out, 0.0)
    mu = jnp.mean(out, axis=0, keepdims=True)
    var = jnp.mean((out - mu) ** 2, axis=0, keepdims=True)
    out = g_ref[...] * (out - mu) / jnp.sqrt(var + 1e-5) + be_ref[...]
    return h + out


def _l0_body(h_ref, parts_ref, cntt_ref, w_ref, b_ref, g_ref, be_ref,
             o_ref, rinv_ref):
    cnt = jnp.sum(cntt_ref[...], axis=1, keepdims=True)
    rinv = 1.0 / jnp.maximum(cnt, 1.0)
    rinv_ref[...] = rinv
    tot = parts_ref[0] + parts_ref[1]
    o_ref[...] = _layer_core(h_ref[...], tot, rinv[0:N],
                             w_ref, b_ref, g_ref, be_ref)


def _lmid_body(h_ref, parts_ref, rinv_ref, w_ref, b_ref, g_ref, be_ref,
               o_ref):
    tot = parts_ref[0] + parts_ref[1]
    o_ref[...] = _layer_core(h_ref[...], tot, rinv_ref[0:N],
                             w_ref, b_ref, g_ref, be_ref)


def _l1_body(h_ref, parts_ref, rinv_ref, w_ref, b_ref, g_ref, be_ref,
             wa_ref, ba_ref, o_ref, s_ref):
    tot = parts_ref[0] + parts_ref[1]
    hn = _layer_core(h_ref[...], tot, rinv_ref[0:N],
                     w_ref, b_ref, g_ref, be_ref)
    o_ref[...] = hn
    z = (jnp.dot(hn, wa_ref[...], preferred_element_type=jnp.float32)
         + ba_ref[...]) / SIGMA
    m = jnp.max(z, axis=-1, keepdims=True)
    ez = jnp.exp(z - m)
    s_ref[...] = ez / jnp.sum(ez, axis=-1, keepdims=True)


def _l3_body(h_ref, parts_ref, rinv_ref, w_ref, b_ref, g_ref, be_ref,
             w0_ref, b0_ref, w1_ref, b1_ref, w2_ref, b2_ref, logits_ref):
    tot = parts_ref[0] + parts_ref[1]
    hn = _layer_core(h_ref[...], tot, rinv_ref[0:N],
                     w_ref, b_ref, g_ref, be_ref)
    hg = jnp.mean(hn, axis=0, keepdims=True)
    z = jnp.maximum(
        jnp.dot(hg, w0_ref[...], preferred_element_type=jnp.float32)
        + b0_ref[...], 0.0)
    z = jnp.maximum(
        jnp.dot(z, w1_ref[...], preferred_element_type=jnp.float32)
        + b1_ref[...], 0.0)
    logits_ref[...] = (
        jnp.dot(z, w2_ref[...], preferred_element_type=jnp.float32)
        + b2_ref[...])


_f32 = jnp.float32
_emb_tc = pl.pallas_call(
    _emb_body, out_shape=jax.ShapeDtypeStruct((N, D), _f32))
_l0_tc = pl.pallas_call(
    _l0_body,
    out_shape=[jax.ShapeDtypeStruct((N, D), _f32),
               jax.ShapeDtypeStruct((NPAD, 1), _f32)])
_lmid_tc = pl.pallas_call(
    _lmid_body, out_shape=jax.ShapeDtypeStruct((N, D), _f32))
_l1_tc = pl.pallas_call(
    _l1_body,
    out_shape=[jax.ShapeDtypeStruct((N, D), _f32),
               jax.ShapeDtypeStruct((N, 32), _f32)])
_l3_tc = pl.pallas_call(
    _l3_body, out_shape=jax.ShapeDtypeStruct((1, 10), _f32))


def kernel(x, e, edge_index, params):
    del e
    src = edge_index[0].astype(jnp.int32)
    dst = edge_index[1].astype(jnp.int32)
    pad = EPAD - E
    srcg = jnp.concatenate([src, jnp.zeros((pad,), jnp.int32)])
    dstg = jnp.concatenate([dst, jnp.full((pad,), N, jnp.int32)])
    srcg = srcg.reshape(NW, NCHUNK, CHUNK)
    dstg = dstg.reshape(NW, NCHUNK, CHUNK)
    zrows = jnp.zeros((CHUNK, D), _f32)
    zcnt = jnp.zeros((NPAD,), _f32)

    p = params

    def r2(v):
        return v.reshape(1, -1)

    _sc_agg_cnt, _sc_agg = _sc_kernels()

    h = _emb_tc(x, p['emb']['W'], r2(p['emb']['b']))

    parts, cntp = _sc_agg_cnt(h, srcg, dstg, zrows, zcnt)
    cntt = cntp.T
    h, rinv = _l0_tc(h, parts, cntt, p['l0']['W'], r2(p['l0']['b']),
                     r2(p['l0']['gamma']), r2(p['l0']['beta']))

    parts = _sc_agg(h, srcg, dstg, zrows)
    h, s = _l1_tc(h, parts, rinv, p['l1']['W'], r2(p['l1']['b']),
                  r2(p['l1']['gamma']), r2(p['l1']['beta']),
                  p['assign']['W'], r2(p['assign']['b']))

    parts = _sc_agg(h, srcg, dstg, zrows)
    h = _lmid_tc(h, parts, rinv, p['l2']['W'], r2(p['l2']['b']),
                 r2(p['l2']['gamma']), r2(p['l2']['beta']))

    parts = _sc_agg(h, srcg, dstg, zrows)
    logits = _l3_tc(h, parts, rinv, p['l3']['W'], r2(p['l3']['b']),
                    r2(p['l3']['gamma']), r2(p['l3']['beta']),
                    p['mlp0']['W'], r2(p['mlp0']['b']),
                    p['mlp1']['W'], r2(p['mlp1']['b']),
                    p['mlp2']['W'], r2(p['mlp2']['b']))
    return (logits, s)
